# baseline (speedup 1.0000x reference)
import functools

import jax
import jax.numpy as jnp
from jax import lax
from jax.experimental import pallas as pl
from jax.experimental.pallas import tpu as pltpu
from jax.experimental.pallas import tpu_sc as plsc

NUM_EMBEDDINGS = 1000000
D = 64
BATCH = 16384
HIST = 50

NC = 2
NS = 16
NW = NC * NS
NBT = BATCH // 128
BT_PER_W = NBT // NW
NU = HIST * 2


def _emb_body(idx_hbm, table_hbm, out_hbm, idx_v,
              gbuf0, gbuf1, tbuf0, tbuf1, gs0, gs1, ws0, ws1):
    gbufs = (gbuf0, gbuf1)
    tbufs = (tbuf0, tbuf1)
    gsems = (gs0, gs1)
    wsems = (ws0, ws1)

    wid = lax.axis_index("s") * NC + lax.axis_index("c")
    bt0 = wid * BT_PER_W

    pltpu.sync_copy(idx_hbm.at[:, pl.ds(bt0, BT_PER_W), :], idx_v)

    def gdesc(s, h, jj, btj):
        return pltpu.make_async_copy(
            table_hbm.at[idx_v.at[h, 2 * jj + btj]],
            gbufs[s].at[pl.ds(btj * 128, 128)], gsems[s])

    def wdesc(s, h, jj, btj):
        return pltpu.make_async_copy(
            tbufs[s].at[btj, :, :, pl.ds(0, 128)],
            out_hbm.at[h, :, bt0 + 2 * jj + btj, :, :], wsems[s])

    gdesc(0, 0, 0, 0).start()
    gdesc(0, 0, 0, 1).start()

    @pl.loop(0, NU, step=2)
    def _round(g):
        h = g // 2
        for j in range(2):
            u = g + j
            gdesc(j, h, j, 0).wait()
            gdesc(j, h, j, 1).wait()

            nh = h + j
            njj = 1 - j

            @pl.when(u + 1 < NU)
            def _():
                gdesc(1 - j, nh, njj, 0).start()
                gdesc(1 - j, nh, njj, 1).start()

            @pl.when(u >= 2)
            def _():
                wdesc(j, h - 1, j, 0).wait()
                wdesc(j, h - 1, j, 1).wait()

            lanes = lax.iota(jnp.int32, 16)
            dsub_vec = lanes % 8
            hi = lanes // 8
            @pl.loop(0, 128)
            def _t(bsub):
                bsplat = jnp.full((16,), bsub, jnp.int32)
                for jj2 in range(2):
                    jsplat = jnp.full((16,), jj2, jnp.int32)
                    for c in range(4):
                        v = gbufs[j][jj2 * 128 + bsub, pl.ds(16 * c, 16)]
                        plsc.store_scatter(
                            tbufs[j], [jsplat, 2 * c + hi, dsub_vec, bsplat],
                            v)

            wdesc(j, h, j, 0).start()
            wdesc(j, h, j, 1).start()

    for btj in range(2):
        wdesc(0, HIST - 1, 0, btj).wait()
        wdesc(1, HIST - 1, 1, btj).wait()


VTF = NUM_EMBEDDINGS // 128
NBLK = VTF // NW + 1


def _wt_body(wt_hbm, wtail_hbm, wlin_hbm,
             bb0, bb1, rb0, rb1, tb, gs0, gs1, ws0, ws1):
    bbs = (bb0, bb1)
    rbs = (rb0, rb1)
    gsems = (gs0, gs1)
    wsems = (ws0, ws1)
    wid = lax.axis_index("s") * NC + lax.axis_index("c")

    @pl.when(wid == 0)
    def _():
        pltpu.sync_copy(wtail_hbm, tb)
        pltpu.sync_copy(tb, wlin_hbm.at[pl.ds(VTF * 8, 4)])

    def gdesc(s, vt):
        return pltpu.make_async_copy(
            wt_hbm.at[:, pl.ds(vt * 128, 128)], bbs[s], gsems[s])

    def wdesc(s, vt):
        return pltpu.make_async_copy(
            rbs[s].at[:, :, pl.ds(0, 128)],
            wlin_hbm.at[pl.ds(vt * 8, 8)], wsems[s])

    gdesc(0, wid).start()
    lanes = lax.iota(jnp.int32, 16)

    @pl.loop(0, NBLK, step=2)
    def _blk(g):
        for s in range(2):
            k = g + s
            vt = wid + NW * k

            @pl.when(vt < VTF)
            def _():
                gdesc(s, vt).wait()
                nvt = wid + NW * (k + 1)

                @pl.when(nvt < VTF)
                def _():
                    gdesc(1 - s, nvt).start()

                @pl.when(k >= 2)
                def _():
                    wdesc(s, wid + NW * (k - 2)).wait()

                rvec = (lanes // 2) % 8
                mbase = (lanes % 2) * 64

                @pl.loop(0, D)
                def _t(d):
                    mvec = mbase + d
                    for c in range(8):
                        v = bbs[s][d, pl.ds(16 * c, 16)]
                        plsc.store_scatter(
                            rbs[s],
                            [jnp.full((16,), c, jnp.int32), rvec, mvec], v)

                wdesc(s, vt).start()

    wdesc(0, wid).wait()
    wdesc(1, wid).wait()


_wt = functools.partial(
    pl.kernel,
    out_type=jax.ShapeDtypeStruct((NUM_EMBEDDINGS // 16, 8, 128), jnp.float32),
    mesh=plsc.VectorSubcoreMesh(core_axis_name="c", subcore_axis_name="s"),
    compiler_params=pltpu.CompilerParams(use_tc_tiling_on_sc=True,
                                         needs_layout_passes=False),
    scratch_types=(
        [pltpu.VMEM((D, 128), jnp.float32) for _ in range(2)]
        + [pltpu.VMEM((8, 8, 129), jnp.float32) for _ in range(2)]
        + [pltpu.VMEM((4, 8, 128), jnp.float32)]
        + [pltpu.SemaphoreType.DMA for _ in range(4)]
    ),
)(_wt_body)


_emb = functools.partial(
    pl.kernel,
    out_type=jax.ShapeDtypeStruct((HIST, 8, NBT, 8, 128), jnp.float32),
    mesh=plsc.VectorSubcoreMesh(core_axis_name="c", subcore_axis_name="s"),
    compiler_params=pltpu.CompilerParams(use_tc_tiling_on_sc=False,
                                         needs_layout_passes=False),
    scratch_types=(
        [pltpu.VMEM((HIST, BT_PER_W, 128), jnp.int32)]
        + [pltpu.VMEM((256, D), jnp.float32) for _ in range(2)]
        + [pltpu.VMEM((2, 8, 8, 129), jnp.float32) for _ in range(2)]
        + [pltpu.SemaphoreType.DMA for _ in range(4)]
    ),
)(_emb_body)


@jax.jit
def kernel(x, weight):
    idx = x.T.reshape(HIST, NBT, 128).astype(jnp.int32)
    wlin = _wt(weight.T, weight[VTF * 128:].reshape(4, 8, 128))
    out5 = _emb(idx, wlin.reshape(NUM_EMBEDDINGS, D))
    return out5.transpose(2, 4, 0, 1, 3).reshape(BATCH, HIST, D)


# speedup vs baseline: 3.3753x; 3.3753x over previous
import functools

import jax
import jax.numpy as jnp
from jax import lax
from jax.experimental import pallas as pl
from jax.experimental.pallas import tpu as pltpu
from jax.experimental.pallas import tpu_sc as plsc

NUM_EMBEDDINGS = 1000000
D = 64
BATCH = 16384
HIST = 50

NC = 2
NS = 16
NW = NC * NS
NBT = BATCH // 128
BT_PER_W = NBT // NW
NU = HIST * 2


def _emb_body(idx_hbm, table_hbm, out_hbm, idx_v,
              gbuf0, gbuf1, tbuf0, tbuf1, gs0, gs1, ws0, ws1):
    gbufs = (gbuf0, gbuf1)
    tbufs = (tbuf0, tbuf1)
    gsems = (gs0, gs1)
    wsems = (ws0, ws1)

    wid = lax.axis_index("s") * NC + lax.axis_index("c")
    bt0 = wid * BT_PER_W

    pltpu.sync_copy(idx_hbm.at[:, pl.ds(bt0, BT_PER_W), :], idx_v)

    def gdesc(s, h, jj, btj):
        return pltpu.make_async_copy(
            table_hbm.at[idx_v.at[h, 2 * jj + btj]],
            gbufs[s].at[pl.ds(btj * 128, 128)], gsems[s])

    def wdesc(s, h, jj, btj):
        return pltpu.make_async_copy(
            tbufs[s].at[btj, :, :, pl.ds(0, 128)],
            out_hbm.at[h, :, bt0 + 2 * jj + btj, :, :], wsems[s])

    gdesc(0, 0, 0, 0).start()
    gdesc(0, 0, 0, 1).start()

    @pl.loop(0, NU, step=2)
    def _round(g):
        h = g // 2
        for j in range(2):
            u = g + j
            gdesc(j, h, j, 0).wait()
            gdesc(j, h, j, 1).wait()

            nh = h + j
            njj = 1 - j

            @pl.when(u + 1 < NU)
            def _():
                gdesc(1 - j, nh, njj, 0).start()
                gdesc(1 - j, nh, njj, 1).start()

            @pl.when(u >= 2)
            def _():
                wdesc(j, h - 1, j, 0).wait()
                wdesc(j, h - 1, j, 1).wait()

            lanes = lax.iota(jnp.int32, 16)
            dsub_vec = lanes % 8
            hi = lanes // 8
            @functools.partial(plsc.parallel_loop, 0, 128, unroll=2)
            def _t(bsub):
                bsplat = jnp.full((16,), bsub, jnp.int32)
                for jj2 in range(2):
                    jsplat = jnp.full((16,), jj2, jnp.int32)
                    for c in range(4):
                        v = gbufs[j][jj2 * 128 + bsub, pl.ds(16 * c, 16)]
                        plsc.store_scatter(
                            tbufs[j], [jsplat, 2 * c + hi, dsub_vec, bsplat],
                            v)

            wdesc(j, h, j, 0).start()
            wdesc(j, h, j, 1).start()

    for btj in range(2):
        wdesc(0, HIST - 1, 0, btj).wait()
        wdesc(1, HIST - 1, 1, btj).wait()


VTF = NUM_EMBEDDINGS // 128
NBLK = VTF // NW + 1


def _wt_body(wt_hbm, wtail_hbm, wlin_hbm,
             bb0, bb1, rb0, rb1, tb, gs0, gs1, ws0, ws1):
    bbs = (bb0, bb1)
    rbs = (rb0, rb1)
    gsems = (gs0, gs1)
    wsems = (ws0, ws1)
    wid = lax.axis_index("s") * NC + lax.axis_index("c")

    @pl.when(wid == 0)
    def _():
        pltpu.sync_copy(wtail_hbm, tb)
        pltpu.sync_copy(tb, wlin_hbm.at[pl.ds(VTF * 8, 4)])

    def gdesc(s, vt):
        return pltpu.make_async_copy(
            wt_hbm.at[:, pl.ds(vt * 128, 128)], bbs[s], gsems[s])

    def wdesc(s, vt):
        return pltpu.make_async_copy(
            rbs[s].at[:, :, pl.ds(0, 128)],
            wlin_hbm.at[pl.ds(vt * 8, 8)], wsems[s])

    gdesc(0, wid).start()
    lanes = lax.iota(jnp.int32, 16)

    @pl.loop(0, NBLK, step=2)
    def _blk(g):
        for s in range(2):
            k = g + s
            vt = wid + NW * k

            @pl.when(vt < VTF)
            def _():
                gdesc(s, vt).wait()
                nvt = wid + NW * (k + 1)

                @pl.when(nvt < VTF)
                def _():
                    gdesc(1 - s, nvt).start()

                @pl.when(k >= 2)
                def _():
                    wdesc(s, wid + NW * (k - 2)).wait()

                rvec = (lanes // 2) % 8
                mbase = (lanes % 2) * 64

                @functools.partial(plsc.parallel_loop, 0, D, unroll=2)
                def _t(d):
                    mvec = mbase + d
                    for c in range(8):
                        v = bbs[s][d, pl.ds(16 * c, 16)]
                        plsc.store_scatter(
                            rbs[s],
                            [jnp.full((16,), c, jnp.int32), rvec, mvec], v)

                wdesc(s, vt).start()

    wdesc(0, wid).wait()
    wdesc(1, wid).wait()


_wt = functools.partial(
    pl.kernel,
    out_type=jax.ShapeDtypeStruct((NUM_EMBEDDINGS // 16, 8, 128), jnp.float32),
    mesh=plsc.VectorSubcoreMesh(core_axis_name="c", subcore_axis_name="s"),
    compiler_params=pltpu.CompilerParams(use_tc_tiling_on_sc=True,
                                         needs_layout_passes=False),
    scratch_types=(
        [pltpu.VMEM((D, 128), jnp.float32) for _ in range(2)]
        + [pltpu.VMEM((8, 8, 129), jnp.float32) for _ in range(2)]
        + [pltpu.VMEM((4, 8, 128), jnp.float32)]
        + [pltpu.SemaphoreType.DMA for _ in range(4)]
    ),
)(_wt_body)


_emb = functools.partial(
    pl.kernel,
    out_type=jax.ShapeDtypeStruct((HIST, 8, NBT, 8, 128), jnp.float32),
    mesh=plsc.VectorSubcoreMesh(core_axis_name="c", subcore_axis_name="s"),
    compiler_params=pltpu.CompilerParams(use_tc_tiling_on_sc=False,
                                         needs_layout_passes=False),
    scratch_types=(
        [pltpu.VMEM((HIST, BT_PER_W, 128), jnp.int32)]
        + [pltpu.VMEM((256, D), jnp.float32) for _ in range(2)]
        + [pltpu.VMEM((2, 8, 8, 129), jnp.float32) for _ in range(2)]
        + [pltpu.SemaphoreType.DMA for _ in range(4)]
    ),
)(_emb_body)


@jax.jit
def kernel(x, weight):
    idx = x.T.reshape(HIST, NBT, 128).astype(jnp.int32)
    wlin = _wt(weight.T, weight[VTF * 128:].reshape(4, 8, 128))
    out5 = _emb(idx, wlin.reshape(NUM_EMBEDDINGS, D))
    return out5.transpose(2, 4, 0, 1, 3).reshape(BATCH, HIST, D)
